# baseline (speedup 1.0000x reference)
import functools

import jax
import jax.numpy as jnp
from jax import lax
from jax.experimental import pallas as pl
from jax.experimental.pallas import tpu as pltpu
from jax.experimental.pallas import tpu_sc as plsc

N = 50000
E = 300000
D = 128
H = 128
C = 16
G = 64

NS = 16
K = 128
NB = 147
SHARE = NB * K
EPAD = NS * SHARE
CN = 12512
CAP = 12544
STRIPE = CAP // NS
DUMMY = CN
LASTF = CN - 15 * STRIPE
LASTT = (N - 3 * CN) - 15 * STRIPE
SEG = 2352
NSEG = SHARE // SEG
CCAP = SEG + K
HR = 512
BN = 2000
GRID = N // BN

_mesh = plsc.VectorSubcoreMesh(core_axis_name="c", subcore_axis_name="s")


HSLOTS = HR * D
HSHARE = EPAD // 32


@functools.partial(
    pl.kernel,
    out_type=jax.ShapeDtypeStruct((32 * HSLOTS,), jnp.float32),
    mesh=_mesh,
    scratch_types=[
        pltpu.VMEM((HSHARE,), jnp.int32),
        pltpu.VMEM((HSLOTS,), jnp.float32),
    ],
    compiler_params=pltpu.CompilerParams(needs_layout_passes=False),
)
def _sc_degree(dst_hbm, part_hbm, dstv, hist):
    s = lax.axis_index("s")
    c = lax.axis_index("c")
    w = c * NS + s
    pltpu.sync_copy(dst_hbm.at[pl.ds(w * HSHARE, HSHARE)], dstv)

    z16 = jnp.zeros((16,), jnp.float32)

    def zrow(r, _):
        hist[pl.ds(r * 16, 16)] = z16
        return 0

    lax.fori_loop(0, HSLOTS // 16, zrow, 0)

    ones16 = jnp.ones((16,), jnp.float32)

    def acc_body(b, _):
        dv = dstv[pl.ds(b * 16, 16)]
        plsc.addupdate_scatter(hist, [dv], ones16)
        return 0

    lax.fori_loop(0, HSHARE // 16, acc_body, 0)

    pltpu.sync_copy(hist, part_hbm.at[pl.ds(w * HSLOTS, HSLOTS)])


@functools.partial(
    pl.kernel,
    out_type=jax.ShapeDtypeStruct((N, D), jnp.float32),
    mesh=_mesh,
    scratch_types=[
        pltpu.VMEM((SEG,), jnp.int32),
        pltpu.VMEM((SEG,), jnp.int32),
        pltpu.VMEM((CCAP,), jnp.int32),
        pltpu.VMEM((CCAP,), jnp.int32),
        pltpu.VMEM((1, K), jnp.int32),
        pltpu.VMEM((K, D), jnp.float32),
        pltpu.VMEM_SHARED((CAP, D), jnp.float32),
    ],
    compiler_params=pltpu.CompilerParams(needs_layout_passes=False),
)
def _sc_scatter(g_hbm, src_hbm, dst_hbm, y_hbm, srcs, dsts, csrc, cdst,
                sidx, rows, acc):
    s = lax.axis_index("s")
    c = lax.axis_index("c")
    tbase = s * SHARE

    z16 = jnp.zeros((16,), jnp.float32)
    zi16 = jnp.zeros((16,), jnp.int32)
    dummy16 = jnp.full((16,), DUMMY, jnp.int32)

    for i in range(2):
        base = (c * 2 + i) * CN

        def zrow(r, _):
            for j in range(8):
                rows[r, pl.ds(j * 16, 16)] = z16
            return 0

        lax.fori_loop(0, K, zrow, 0)
        for k in range(6):
            pltpu.sync_copy(rows, acc.at[pl.ds(s * STRIPE + k * K, K)])
        pltpu.sync_copy(rows.at[pl.ds(0, 16)],
                        acc.at[pl.ds(s * STRIPE + 6 * K, 16)])
        plsc.subcore_barrier()

        for seg in range(NSEG):
            pltpu.sync_copy(src_hbm.at[pl.ds(tbase + seg * SEG, SEG)], srcs)
            pltpu.sync_copy(dst_hbm.at[pl.ds(tbase + seg * SEG, SEG)], dsts)

            def compact(v, off):
                dv = dsts[pl.ds(v * 16, 16)]
                sv = srcs[pl.ds(v * 16, 16)]
                ok = (dv >= base) & (dv < base + CN)
                plsc.store_compressed(cdst.at[pl.ds(off, 16)], dv - base,
                                      mask=ok)
                plsc.store_compressed(csrc.at[pl.ds(off, 16)], sv, mask=ok)
                pc = plsc.all_reduce_population_count(ok)
                return off + lax.reduce_max(pc, (0,))

            off = lax.fori_loop(0, SEG // 16, compact, jnp.int32(0))
            rup = ((off + K - 1) >> 7) << 7
            for j in range(8):
                @pl.when(off + j * 16 < rup)
                def _():
                    cdst[pl.ds(off + j * 16, 16)] = dummy16
                    csrc[pl.ds(off + j * 16, 16)] = zi16

            def drain(b, _):
                for j in range(8):
                    sidx[0, pl.ds(j * 16, 16)] = cdst[pl.ds(b * K + j * 16, 16)]
                pltpu.sync_copy(g_hbm.at[csrc.at[pl.ds(b * K, K)]], rows)
                pltpu.sync_copy(rows, acc.at[sidx.at[0]], add=True)
                return 0

            lax.fori_loop(0, rup >> 7, drain, 0)
        plsc.subcore_barrier()

        @pl.when(s < NS - 1)
        def _():
            pltpu.sync_copy(acc.at[pl.ds(s * STRIPE, STRIPE)],
                            y_hbm.at[pl.ds(base + s * STRIPE, STRIPE)])

        @pl.when(s == NS - 1)
        def _():
            off = (NS - 1) * STRIPE
            if i == 0:
                pltpu.sync_copy(acc.at[pl.ds(off, LASTF)],
                                y_hbm.at[pl.ds(base + off, LASTF)])
            else:
                @pl.when(c == 0)
                def _():
                    pltpu.sync_copy(acc.at[pl.ds(off, LASTF)],
                                    y_hbm.at[pl.ds(base + off, LASTF)])

                @pl.when(c == 1)
                def _():
                    pltpu.sync_copy(acc.at[pl.ds(off, LASTT)],
                                    y_hbm.at[pl.ds(base + off, LASTT)])

        plsc.subcore_barrier()


def _dinv_body(p_ref, dinv_ref):
    deg = jnp.sum(p_ref[...], axis=0) + 1.0
    dinv_ref[...] = lax.rsqrt(deg)


def _dinv(part_flat):
    part = part_flat.reshape(32, HR, D)
    out = pl.pallas_call(
        _dinv_body,
        out_shape=jax.ShapeDtypeStruct((HR, D), jnp.float32),
    )(part)
    return out.reshape(-1)[:N].reshape(N, 1)


def _prep_body(x_ref, dn_ref, dx_ref, wn_ref, wx_ref, gn_ref, gx_ref):
    x = x_ref[...]
    gn_ref[...] = dn_ref[...] * jnp.dot(
        x, wn_ref[...], preferred_element_type=jnp.float32)
    gx_ref[...] = dx_ref[...] * jnp.dot(
        x, wx_ref[...], preferred_element_type=jnp.float32)


def _prep(x, dinv_n, dinv_x, Wn, Wx):
    blk = lambda i: (i, 0)
    zero = lambda i: (0, 0)
    return pl.pallas_call(
        _prep_body,
        grid=(GRID,),
        in_specs=[
            pl.BlockSpec((BN, D), blk),
            pl.BlockSpec((BN, 1), blk),
            pl.BlockSpec((BN, 1), blk),
            pl.BlockSpec((D, H), zero),
            pl.BlockSpec((D, H), zero),
        ],
        out_specs=[pl.BlockSpec((BN, H), blk), pl.BlockSpec((BN, H), blk)],
        out_shape=[
            jax.ShapeDtypeStruct((N, H), jnp.float32),
            jax.ShapeDtypeStruct((N, H), jnp.float32),
        ],
    )(x, dinv_n, dinv_x, Wn, Wx)


def _combine0_body(yn_ref, gn_ref, yx_ref, gx_ref, dn_ref, dx_ref,
                   bn_ref, bx_ref, h_ref):
    hn = dn_ref[...] * (yn_ref[...] + gn_ref[...]) + bn_ref[...]
    hx = dx_ref[...] * (yx_ref[...] + gx_ref[...]) + bx_ref[...]
    h_ref[...] = jnp.maximum(hn + hx, 0.0)


def _combine0(yn, gn, yx, gx, dinv_n, dinv_x, bn, bx):
    blk = lambda i: (i, 0)
    zero = lambda i: (0, 0)
    return pl.pallas_call(
        _combine0_body,
        grid=(GRID,),
        in_specs=[
            pl.BlockSpec((BN, H), blk),
            pl.BlockSpec((BN, H), blk),
            pl.BlockSpec((BN, H), blk),
            pl.BlockSpec((BN, H), blk),
            pl.BlockSpec((BN, 1), blk),
            pl.BlockSpec((BN, 1), blk),
            pl.BlockSpec((1, H), zero),
            pl.BlockSpec((1, H), zero),
        ],
        out_specs=pl.BlockSpec((BN, H), blk),
        out_shape=jax.ShapeDtypeStruct((N, H), jnp.float32),
    )(yn, gn, yx, gx, dinv_n, dinv_x, bn.reshape(1, H), bx.reshape(1, H))


def _final_body(yn_ref, gn_ref, yx_ref, gx_ref, dn_ref, dx_ref,
                bn_ref, bx_ref, seg_ref, lw_ref, lb_ref, out_ref,
                sums, cnt):
    i = pl.program_id(0)

    @pl.when(i == 0)
    def _():
        sums[...] = jnp.zeros((G, H), jnp.float32)
        cnt[...] = jnp.zeros((G, 1), jnp.float32)

    hn = dn_ref[...] * (yn_ref[...] + gn_ref[...]) + bn_ref[...]
    hx = dx_ref[...] * (yx_ref[...] + gx_ref[...]) + bx_ref[...]
    h2 = hn + hx
    seg = seg_ref[0, 0, :]
    oh = (lax.broadcasted_iota(jnp.int32, (G, BN), 0)
          == seg[None, :]).astype(jnp.float32)
    sums[...] += jnp.dot(oh, h2, preferred_element_type=jnp.float32)
    cnt[...] += jnp.dot(oh, jnp.ones((BN, 1), jnp.float32),
                        preferred_element_type=jnp.float32)

    @pl.when(i == GRID - 1)
    def _():
        pooled = sums[...] / jnp.maximum(cnt[...], 1.0)
        out_ref[...] = jnp.dot(pooled, lw_ref[...],
                               preferred_element_type=jnp.float32) + lb_ref[...]


def _final(yn, gn, yx, gx, dinv_n, dinv_x, bn, bx, seg3, lin_W, lin_b):
    blk = lambda i: (i, 0)
    zero = lambda i: (0, 0)
    return pl.pallas_call(
        _final_body,
        grid=(GRID,),
        in_specs=[
            pl.BlockSpec((BN, H), blk),
            pl.BlockSpec((BN, H), blk),
            pl.BlockSpec((BN, H), blk),
            pl.BlockSpec((BN, H), blk),
            pl.BlockSpec((BN, 1), blk),
            pl.BlockSpec((BN, 1), blk),
            pl.BlockSpec((1, H), zero),
            pl.BlockSpec((1, H), zero),
            pl.BlockSpec((1, 1, BN), lambda i: (i, 0, 0)),
            pl.BlockSpec((H, C), zero),
            pl.BlockSpec((1, C), zero),
        ],
        out_specs=pl.BlockSpec((G, C), zero),
        out_shape=jax.ShapeDtypeStruct((G, C), jnp.float32),
        scratch_shapes=[
            pltpu.VMEM((G, H), jnp.float32),
            pltpu.VMEM((G, 1), jnp.float32),
        ],
    )(yn, gn, yx, gx, dinv_n, dinv_x, bn.reshape(1, H), bx.reshape(1, H),
      seg3, lin_W, lin_b.reshape(1, C))


def _pad_edges(ei):
    src = jnp.concatenate(
        [ei[0].astype(jnp.int32), jnp.zeros((EPAD - E,), jnp.int32)])
    dst = jnp.concatenate(
        [ei[1].astype(jnp.int32), jnp.full((EPAD - E,), N, jnp.int32)])
    return src, dst


def kernel(x_traj_point, edge_index_near, edge_index_next, batch_traj_point,
           W0_near, b0_near, W0_next, b0_next,
           W1_near, b1_near, W1_next, b1_next,
           lin_W, lin_b):
    src_n, dst_n = _pad_edges(edge_index_near)
    src_x, dst_x = _pad_edges(edge_index_next)
    seg3 = batch_traj_point.astype(jnp.int32).reshape(GRID, 1, BN)

    dinv_n = _dinv(_sc_degree(dst_n))
    dinv_x = _dinv(_sc_degree(dst_x))

    g0n, g0x = _prep(x_traj_point, dinv_n, dinv_x, W0_near, W0_next)
    y0n = _sc_scatter(g0n, src_n, dst_n)
    y0x = _sc_scatter(g0x, src_x, dst_x)
    h = _combine0(y0n, g0n, y0x, g0x, dinv_n, dinv_x, b0_near, b0_next)

    g1n, g1x = _prep(h, dinv_n, dinv_x, W1_near, W1_next)
    y1n = _sc_scatter(g1n, src_n, dst_n)
    y1x = _sc_scatter(g1x, src_x, dst_x)
    return _final(y1n, g1n, y1x, g1x, dinv_n, dinv_x, b1_near, b1_next,
                  seg3, lin_W, lin_b)


# speedup vs baseline: 1.5482x; 1.5482x over previous
import functools

import jax
import jax.numpy as jnp
from jax import lax
from jax.experimental import pallas as pl
from jax.experimental.pallas import tpu as pltpu
from jax.experimental.pallas import tpu_sc as plsc

N = 50000
E = 300000
D = 128
H = 128
C = 16
G = 64

NS = 16
K = 128
NB = 147
SHARE = NB * K
EPAD = NS * SHARE
CN = 12512
CAP = 12544
STRIPE = CAP // NS
DUMMY = CN
LASTF = CN - 15 * STRIPE
LASTT = (N - 3 * CN) - 15 * STRIPE
KB = 112
NBB = SHARE // KB
HR = 512
BN = 2000
GRID = N // BN

_mesh = plsc.VectorSubcoreMesh(core_axis_name="c", subcore_axis_name="s")


HSLOTS = HR * D
HSHARE = EPAD // 32


@functools.partial(
    pl.kernel,
    out_type=jax.ShapeDtypeStruct((32 * HSLOTS,), jnp.float32),
    mesh=_mesh,
    scratch_types=[
        pltpu.VMEM((HSHARE,), jnp.int32),
        pltpu.VMEM((HSLOTS,), jnp.float32),
    ],
    compiler_params=pltpu.CompilerParams(needs_layout_passes=False),
)
def _sc_degree(dst_hbm, part_hbm, dstv, hist):
    s = lax.axis_index("s")
    c = lax.axis_index("c")
    w = c * NS + s
    pltpu.sync_copy(dst_hbm.at[pl.ds(w * HSHARE, HSHARE)], dstv)

    z16 = jnp.zeros((16,), jnp.float32)

    def zrow(r, _):
        hist[pl.ds(r * 16, 16)] = z16
        return 0

    lax.fori_loop(0, HSLOTS // 16, zrow, 0)

    ones16 = jnp.ones((16,), jnp.float32)

    def acc_body(b, _):
        dv = dstv[pl.ds(b * 16, 16)]
        plsc.addupdate_scatter(hist, [dv], ones16)
        return 0

    lax.fori_loop(0, HSHARE // 16, acc_body, 0)

    pltpu.sync_copy(hist, part_hbm.at[pl.ds(w * HSLOTS, HSLOTS)])


@functools.partial(
    pl.kernel,
    out_type=jax.ShapeDtypeStruct((N, D), jnp.float32),
    mesh=_mesh,
    scratch_types=[
        pltpu.VMEM((3, KB), jnp.int32),
        pltpu.VMEM((3, KB), jnp.int32),
        pltpu.VMEM((1, KB), jnp.int32),
        pltpu.VMEM((2, KB, D), jnp.float32),
        pltpu.VMEM_SHARED((CAP, D), jnp.float32),
        pltpu.SemaphoreType.DMA,
        pltpu.SemaphoreType.DMA,
    ],
    compiler_params=pltpu.CompilerParams(needs_layout_passes=False),
)
def _sc_scatter(g_hbm, src_hbm, dst_hbm, y_hbm, srcb, dstb, sidx, rows, acc,
                sem_g, sem_i):
    s = lax.axis_index("s")
    c = lax.axis_index("c")
    tbase = s * SHARE

    z16 = jnp.zeros((16,), jnp.float32)

    for i in range(2):
        base = (c * 2 + i) * CN

        def zrow(r, _):
            for j in range(8):
                rows[0, r, pl.ds(j * 16, 16)] = z16
            return 0

        lax.fori_loop(0, KB, zrow, 0)
        for k in range(7):
            pltpu.sync_copy(rows.at[0], acc.at[pl.ds(s * STRIPE + k * KB, KB)])
        plsc.subcore_barrier()

        pltpu.sync_copy(src_hbm.at[pl.ds(tbase, KB)], srcb.at[0])
        pltpu.sync_copy(dst_hbm.at[pl.ds(tbase, KB)], dstb.at[0])
        pltpu.sync_copy(src_hbm.at[pl.ds(tbase + KB, KB)], srcb.at[1])
        pltpu.sync_copy(dst_hbm.at[pl.ds(tbase + KB, KB)], dstb.at[1])
        pltpu.async_copy(g_hbm.at[srcb.at[0]], rows.at[0], sem_g)

        def batch(b, _):
            p = b % 2
            bi = b % 3
            for j in range(7):
                dv = dstb[bi, pl.ds(j * 16, 16)]
                ok = (dv >= base) & (dv < base + CN)
                sidx[0, pl.ds(j * 16, 16)] = jnp.where(ok, dv - base, DUMMY)
            pltpu.make_async_copy(
                g_hbm.at[srcb.at[bi]], rows.at[p], sem_g).wait()

            @pl.when(b + 1 < NBB)
            def _():
                pltpu.async_copy(
                    g_hbm.at[srcb.at[(b + 1) % 3]], rows.at[1 - p], sem_g)

            pltpu.sync_copy(rows.at[p], acc.at[sidx.at[0]], add=True)

            @pl.when(b + 2 < NBB)
            def _():
                off = tbase + (b + 2) * KB
                ni = (b + 2) % 3
                cp1 = pltpu.async_copy(
                    src_hbm.at[pl.ds(off, KB)], srcb.at[ni], sem_i)
                cp2 = pltpu.async_copy(
                    dst_hbm.at[pl.ds(off, KB)], dstb.at[ni], sem_i)
                cp1.wait()
                cp2.wait()

            return 0

        lax.fori_loop(0, NBB, batch, 0)
        plsc.subcore_barrier()

        @pl.when(s < NS - 1)
        def _():
            pltpu.sync_copy(acc.at[pl.ds(s * STRIPE, STRIPE)],
                            y_hbm.at[pl.ds(base + s * STRIPE, STRIPE)])

        @pl.when(s == NS - 1)
        def _():
            off = (NS - 1) * STRIPE
            if i == 0:
                pltpu.sync_copy(acc.at[pl.ds(off, LASTF)],
                                y_hbm.at[pl.ds(base + off, LASTF)])
            else:
                @pl.when(c == 0)
                def _():
                    pltpu.sync_copy(acc.at[pl.ds(off, LASTF)],
                                    y_hbm.at[pl.ds(base + off, LASTF)])

                @pl.when(c == 1)
                def _():
                    pltpu.sync_copy(acc.at[pl.ds(off, LASTT)],
                                    y_hbm.at[pl.ds(base + off, LASTT)])

        plsc.subcore_barrier()


def _dinv_body(p_ref, dinv_ref):
    deg = jnp.sum(p_ref[...], axis=0) + 1.0
    dinv_ref[...] = lax.rsqrt(deg)


def _dinv(part_flat):
    part = part_flat.reshape(32, HR, D)
    out = pl.pallas_call(
        _dinv_body,
        out_shape=jax.ShapeDtypeStruct((HR, D), jnp.float32),
    )(part)
    return out.reshape(-1)[:N].reshape(N, 1)


def _prep_body(x_ref, dn_ref, dx_ref, wn_ref, wx_ref, gn_ref, gx_ref):
    x = x_ref[...]
    gn_ref[...] = dn_ref[...] * jnp.dot(
        x, wn_ref[...], preferred_element_type=jnp.float32)
    gx_ref[...] = dx_ref[...] * jnp.dot(
        x, wx_ref[...], preferred_element_type=jnp.float32)


def _prep(x, dinv_n, dinv_x, Wn, Wx):
    blk = lambda i: (i, 0)
    zero = lambda i: (0, 0)
    return pl.pallas_call(
        _prep_body,
        grid=(GRID,),
        in_specs=[
            pl.BlockSpec((BN, D), blk),
            pl.BlockSpec((BN, 1), blk),
            pl.BlockSpec((BN, 1), blk),
            pl.BlockSpec((D, H), zero),
            pl.BlockSpec((D, H), zero),
        ],
        out_specs=[pl.BlockSpec((BN, H), blk), pl.BlockSpec((BN, H), blk)],
        out_shape=[
            jax.ShapeDtypeStruct((N, H), jnp.float32),
            jax.ShapeDtypeStruct((N, H), jnp.float32),
        ],
    )(x, dinv_n, dinv_x, Wn, Wx)


def _combine0_body(yn_ref, gn_ref, yx_ref, gx_ref, dn_ref, dx_ref,
                   bn_ref, bx_ref, h_ref):
    hn = dn_ref[...] * (yn_ref[...] + gn_ref[...]) + bn_ref[...]
    hx = dx_ref[...] * (yx_ref[...] + gx_ref[...]) + bx_ref[...]
    h_ref[...] = jnp.maximum(hn + hx, 0.0)


def _combine0(yn, gn, yx, gx, dinv_n, dinv_x, bn, bx):
    blk = lambda i: (i, 0)
    zero = lambda i: (0, 0)
    return pl.pallas_call(
        _combine0_body,
        grid=(GRID,),
        in_specs=[
            pl.BlockSpec((BN, H), blk),
            pl.BlockSpec((BN, H), blk),
            pl.BlockSpec((BN, H), blk),
            pl.BlockSpec((BN, H), blk),
            pl.BlockSpec((BN, 1), blk),
            pl.BlockSpec((BN, 1), blk),
            pl.BlockSpec((1, H), zero),
            pl.BlockSpec((1, H), zero),
        ],
        out_specs=pl.BlockSpec((BN, H), blk),
        out_shape=jax.ShapeDtypeStruct((N, H), jnp.float32),
    )(yn, gn, yx, gx, dinv_n, dinv_x, bn.reshape(1, H), bx.reshape(1, H))


def _final_body(yn_ref, gn_ref, yx_ref, gx_ref, dn_ref, dx_ref,
                bn_ref, bx_ref, seg_ref, lw_ref, lb_ref, out_ref,
                sums, cnt):
    i = pl.program_id(0)

    @pl.when(i == 0)
    def _():
        sums[...] = jnp.zeros((G, H), jnp.float32)
        cnt[...] = jnp.zeros((G, 1), jnp.float32)

    hn = dn_ref[...] * (yn_ref[...] + gn_ref[...]) + bn_ref[...]
    hx = dx_ref[...] * (yx_ref[...] + gx_ref[...]) + bx_ref[...]
    h2 = hn + hx
    seg = seg_ref[0, 0, :]
    oh = (lax.broadcasted_iota(jnp.int32, (G, BN), 0)
          == seg[None, :]).astype(jnp.float32)
    sums[...] += jnp.dot(oh, h2, preferred_element_type=jnp.float32)
    cnt[...] += jnp.dot(oh, jnp.ones((BN, 1), jnp.float32),
                        preferred_element_type=jnp.float32)

    @pl.when(i == GRID - 1)
    def _():
        pooled = sums[...] / jnp.maximum(cnt[...], 1.0)
        out_ref[...] = jnp.dot(pooled, lw_ref[...],
                               preferred_element_type=jnp.float32) + lb_ref[...]


def _final(yn, gn, yx, gx, dinv_n, dinv_x, bn, bx, seg3, lin_W, lin_b):
    blk = lambda i: (i, 0)
    zero = lambda i: (0, 0)
    return pl.pallas_call(
        _final_body,
        grid=(GRID,),
        in_specs=[
            pl.BlockSpec((BN, H), blk),
            pl.BlockSpec((BN, H), blk),
            pl.BlockSpec((BN, H), blk),
            pl.BlockSpec((BN, H), blk),
            pl.BlockSpec((BN, 1), blk),
            pl.BlockSpec((BN, 1), blk),
            pl.BlockSpec((1, H), zero),
            pl.BlockSpec((1, H), zero),
            pl.BlockSpec((1, 1, BN), lambda i: (i, 0, 0)),
            pl.BlockSpec((H, C), zero),
            pl.BlockSpec((1, C), zero),
        ],
        out_specs=pl.BlockSpec((G, C), zero),
        out_shape=jax.ShapeDtypeStruct((G, C), jnp.float32),
        scratch_shapes=[
            pltpu.VMEM((G, H), jnp.float32),
            pltpu.VMEM((G, 1), jnp.float32),
        ],
    )(yn, gn, yx, gx, dinv_n, dinv_x, bn.reshape(1, H), bx.reshape(1, H),
      seg3, lin_W, lin_b.reshape(1, C))


def _pad_edges(ei):
    src = jnp.concatenate(
        [ei[0].astype(jnp.int32), jnp.zeros((EPAD - E,), jnp.int32)])
    dst = jnp.concatenate(
        [ei[1].astype(jnp.int32), jnp.full((EPAD - E,), N, jnp.int32)])
    return src, dst


def kernel(x_traj_point, edge_index_near, edge_index_next, batch_traj_point,
           W0_near, b0_near, W0_next, b0_next,
           W1_near, b1_near, W1_next, b1_next,
           lin_W, lin_b):
    src_n, dst_n = _pad_edges(edge_index_near)
    src_x, dst_x = _pad_edges(edge_index_next)
    seg3 = batch_traj_point.astype(jnp.int32).reshape(GRID, 1, BN)

    dinv_n = _dinv(_sc_degree(dst_n))
    dinv_x = _dinv(_sc_degree(dst_x))

    g0n, g0x = _prep(x_traj_point, dinv_n, dinv_x, W0_near, W0_next)
    y0n = _sc_scatter(g0n, src_n, dst_n)
    y0x = _sc_scatter(g0x, src_x, dst_x)
    h = _combine0(y0n, g0n, y0x, g0x, dinv_n, dinv_x, b0_near, b0_next)

    g1n, g1x = _prep(h, dinv_n, dinv_x, W1_near, W1_next)
    y1n = _sc_scatter(g1n, src_n, dst_n)
    y1x = _sc_scatter(g1x, src_x, dst_x)
    return _final(y1n, g1n, y1x, g1x, dinv_n, dinv_x, b1_near, b1_next,
                  seg3, lin_W, lin_b)
